# Initial kernel scaffold; baseline (speedup 1.0000x reference)
import jax
import jax.numpy as jnp
from jax.experimental import pallas as pl


def kernel(weights, indices):
    raise NotImplementedError("write your pallas kernel here")


# speedup vs baseline: 1.1019x; 1.1019x over previous
import functools

import jax
import jax.numpy as jnp
from jax import lax
from jax.experimental import pallas as pl
from jax.experimental.pallas import tpu as pltpu
from jax.experimental.pallas import tpu_sc as plsc

_INFO = plsc.get_sparse_core_info()
_NC = _INFO.num_cores
_NS = _INFO.num_subcores
_NW = _NC * _NS


def _make_gather(V, D, B):
    assert B % _NW == 0
    b_per_w = B // _NW
    chunk = 1600
    assert b_per_w % chunk == 0
    n_chunks = b_per_w // chunk
    mesh = plsc.VectorSubcoreMesh(core_axis_name="c", subcore_axis_name="s")

    @functools.partial(
        pl.kernel,
        mesh=mesh,
        out_type=jax.ShapeDtypeStruct((B, D), jnp.float32),
        scratch_types=[
            pltpu.VMEM((chunk,), jnp.int32),
            pltpu.VMEM((chunk, D), jnp.float32),
            pltpu.SemaphoreType.DMA,
        ],
        compiler_params=pltpu.CompilerParams(use_tc_tiling_on_sc=False),
    )
    def k(table_hbm, idx_hbm, out_hbm, idx_v, rows_v, sem):
        wid = lax.axis_index("s") * _NC + lax.axis_index("c")
        base = wid * b_per_w

        def body(i, _):
            cb = base + i * chunk
            pltpu.sync_copy(idx_hbm.at[pl.ds(cb, chunk)], idx_v)
            pltpu.async_copy(table_hbm.at[idx_v], rows_v, sem).wait()
            pltpu.sync_copy(rows_v, out_hbm.at[pl.ds(cb, chunk)])
            return 0

        lax.fori_loop(0, n_chunks, body, 0)

    return k


def kernel(weights, indices):
    D = weights.shape[1]
    idx_flat = indices.reshape(-1).astype(jnp.int32)
    out = _make_gather(weights.shape[0], D, idx_flat.shape[0])(weights, idx_flat)
    return out.reshape(indices.shape + (D,))


# speedup vs baseline: 1.1126x; 1.0097x over previous
import functools

import jax
import jax.numpy as jnp
from jax import lax
from jax.experimental import pallas as pl
from jax.experimental.pallas import tpu as pltpu
from jax.experimental.pallas import tpu_sc as plsc

_INFO = plsc.get_sparse_core_info()
_NC = _INFO.num_cores
_NS = _INFO.num_subcores
_NW = _NC * _NS


def _make_gather(V, D, B):
    assert B % _NW == 0
    b_per_w = B // _NW
    chunk = 1600
    assert b_per_w % (2 * chunk) == 0
    n_chunks = b_per_w // chunk
    n_pairs = n_chunks // 2
    mesh = plsc.VectorSubcoreMesh(core_axis_name="c", subcore_axis_name="s")

    @functools.partial(
        pl.kernel,
        mesh=mesh,
        out_type=jax.ShapeDtypeStruct((B, D), jnp.float32),
        scratch_types=[
            pltpu.VMEM((b_per_w,), jnp.int32),
            pltpu.VMEM((chunk, D), jnp.float32),
            pltpu.VMEM((chunk, D), jnp.float32),
            pltpu.SemaphoreType.DMA,
            pltpu.SemaphoreType.DMA,
            pltpu.SemaphoreType.DMA,
            pltpu.SemaphoreType.DMA,
        ],
        compiler_params=pltpu.CompilerParams(use_tc_tiling_on_sc=False),
    )
    def k(table_hbm, idx_hbm, out_hbm, idx_v, rows0, rows1, g0, g1, o0, o1):
        wid = lax.axis_index("s") * _NC + lax.axis_index("c")
        base = wid * b_per_w
        pltpu.sync_copy(idx_hbm.at[pl.ds(base, b_per_w)], idx_v)

        def gather(i, rows, sem):
            pltpu.async_copy(
                table_hbm.at[idx_v.at[pl.ds(i * chunk, chunk)]], rows, sem)

        def store(i, rows, sem):
            pltpu.async_copy(rows, out_hbm.at[pl.ds(base + i * chunk, chunk)], sem)

        def wait_gather(rows, sem):
            pltpu.make_async_copy(table_hbm.at[pl.ds(0, chunk)], rows, sem).wait()

        def wait_store(i, rows, sem):
            pltpu.make_async_copy(
                rows, out_hbm.at[pl.ds(base + i * chunk, chunk)], sem).wait()

        gather(0, rows0, g0)
        gather(1, rows1, g1)

        def body(j, _):
            i0 = 2 * j
            wait_gather(rows0, g0)
            store(i0, rows0, o0)
            wait_store(i0, rows0, o0)
            gather(i0 + 2, rows0, g0)
            wait_gather(rows1, g1)
            store(i0 + 1, rows1, o1)
            wait_store(i0 + 1, rows1, o1)
            gather(i0 + 3, rows1, g1)
            return 0

        lax.fori_loop(0, n_pairs - 1, body, 0)

        i0 = n_chunks - 2
        wait_gather(rows0, g0)
        store(i0, rows0, o0)
        wait_gather(rows1, g1)
        store(i0 + 1, rows1, o1)
        wait_store(i0, rows0, o0)
        wait_store(i0 + 1, rows1, o1)

    return k


def kernel(weights, indices):
    D = weights.shape[1]
    idx_flat = indices.reshape(-1).astype(jnp.int32)
    out = _make_gather(weights.shape[0], D, idx_flat.shape[0])(weights, idx_flat)
    return out.reshape(indices.shape + (D,))
